# Initial kernel scaffold; baseline (speedup 1.0000x reference)
import jax
import jax.numpy as jnp
from jax.experimental import pallas as pl


def kernel(x, edge_attr, recipe, params, edge_index, batch):
    raise NotImplementedError("write your pallas kernel here")


# speedup vs baseline: 1.0581x; 1.0581x over previous
import jax
import jax.numpy as jnp
from jax import lax
from jax.experimental import pallas as pl
from jax.experimental.pallas import tpu as pltpu

N = 50000
E = 800000
HID = 64
HEADS = 4
CH = 16
NG = 8

_BN = 1000
_BE = 4000

_HI = lax.Precision.HIGHEST


def _r(v):
    return v.astype(jnp.bfloat16).astype(jnp.float32)


def _node_frontend_body(x_ref, batch_ref, t0_ref, t1_ref, r8_ref, w8_ref,
                        b_ref, out_ref):
    xb = x_ref[...]
    x8 = _r(xb[:, 0:8])
    i0 = xb[:, 8:9].astype(jnp.int32)
    i1 = xb[:, 9:10].astype(jnp.int32)
    bb = batch_ref[...]
    bn = xb.shape[0]
    acc = jnp.dot(x8, w8_ref[...], preferred_element_type=jnp.float32,
                  precision=_HI)
    oh0 = (lax.broadcasted_iota(jnp.int32, (bn, 64), 1) == i0).astype(jnp.float32)
    acc += jnp.dot(oh0, t0_ref[...], preferred_element_type=jnp.float32,
                   precision=_HI)
    oh1 = (lax.broadcasted_iota(jnp.int32, (bn, 512), 1) == i1).astype(jnp.float32)
    acc += jnp.dot(oh1, t1_ref[...], preferred_element_type=jnp.float32,
                   precision=_HI)
    ohb = (lax.broadcasted_iota(jnp.int32, (bn, NG), 1) == bb).astype(jnp.float32)
    acc += jnp.dot(ohb, r8_ref[...], preferred_element_type=jnp.float32,
                   precision=_HI)
    out_ref[...] = jnp.maximum(acc + b_ref[...], 0.0)


def _node_frontend(x, batch2d, t0, t1, r8, w8, b):
    return pl.pallas_call(
        _node_frontend_body,
        grid=(N // _BN,),
        in_specs=[
            pl.BlockSpec((_BN, 10), lambda i: (i, 0)),
            pl.BlockSpec((_BN, 1), lambda i: (i, 0)),
            pl.BlockSpec((64, 64), lambda i: (0, 0)),
            pl.BlockSpec((512, 64), lambda i: (0, 0)),
            pl.BlockSpec((NG, 64), lambda i: (0, 0)),
            pl.BlockSpec((8, 64), lambda i: (0, 0)),
            pl.BlockSpec((1, 64), lambda i: (0, 0)),
        ],
        out_specs=pl.BlockSpec((_BN, 64), lambda i: (i, 0)),
        out_shape=jax.ShapeDtypeStruct((N, 64), jnp.float32),
    )(x, batch2d, t0, t1, r8, w8, b)


def _edge_frontend_body(ea_ref, m4_ref, te_ref, ce_ref, out_ref):
    eb = ea_ref[...]
    e4 = _r(eb[:, 0:4])
    ei = eb[:, 4:5].astype(jnp.int32)
    be = eb.shape[0]
    acc = jnp.dot(e4, m4_ref[...], preferred_element_type=jnp.float32,
                  precision=_HI)
    ohe = (lax.broadcasted_iota(jnp.int32, (be, 32), 1) == ei).astype(jnp.float32)
    acc += jnp.dot(ohe, te_ref[...], preferred_element_type=jnp.float32,
                   precision=_HI)
    out_ref[...] = acc + ce_ref[...]


def _edge_frontend(edge_attr, m4, te, ce):
    return pl.pallas_call(
        _edge_frontend_body,
        grid=(E // _BE,),
        in_specs=[
            pl.BlockSpec((_BE, 5), lambda i: (i, 0)),
            pl.BlockSpec((4, 12), lambda i: (0, 0)),
            pl.BlockSpec((32, 12), lambda i: (0, 0)),
            pl.BlockSpec((1, 12), lambda i: (0, 0)),
        ],
        out_specs=pl.BlockSpec((_BE, 12), lambda i: (i, 0)),
        out_shape=jax.ShapeDtypeStruct((E, 12), jnp.float32),
    )(edge_attr, m4, te, ce)


def _layer_dense_body(h_ref, bprev_ref, w_ref, asrc_ref, adst_ref, s_ref,
                      out_ref):
    h = jnp.maximum(h_ref[...] + bprev_ref[...], 0.0)
    hl = jnp.dot(h, w_ref[...], preferred_element_type=jnp.float32)
    a_src = jnp.dot(hl * asrc_ref[...], s_ref[...],
                    preferred_element_type=jnp.float32, precision=_HI)
    a_dst = jnp.dot(hl * adst_ref[...], s_ref[...],
                    preferred_element_type=jnp.float32, precision=_HI)
    out_ref[...] = jnp.concatenate([hl, a_src, a_dst], axis=1)


def _layer_dense(h_raw, bprev, w, asrc_row, adst_row, s_mat):
    return pl.pallas_call(
        _layer_dense_body,
        grid=(N // _BN,),
        in_specs=[
            pl.BlockSpec((_BN, 64), lambda i: (i, 0)),
            pl.BlockSpec((1, 64), lambda i: (0, 0)),
            pl.BlockSpec((64, 64), lambda i: (0, 0)),
            pl.BlockSpec((1, 64), lambda i: (0, 0)),
            pl.BlockSpec((1, 64), lambda i: (0, 0)),
            pl.BlockSpec((64, 4), lambda i: (0, 0)),
        ],
        out_specs=pl.BlockSpec((_BN, 72), lambda i: (i, 0)),
        out_shape=jax.ShapeDtypeStruct((N, 72), jnp.float32),
    )(h_raw, bprev, w, asrc_row, adst_row, s_mat)


def _pool_mlp_body(h_ref, batch_ref, b2_ref, wr1_ref, br1_ref, wr2_ref,
                   br2_ref, out_ref, g_acc):
    j = pl.program_id(0)
    h = jnp.maximum(h_ref[...] + b2_ref[...], 0.0)
    bb = batch_ref[...]
    bn = h.shape[0]
    oh = (lax.broadcasted_iota(jnp.int32, (bn, NG), 1) == bb).astype(jnp.float32)
    contrib = lax.dot_general(oh, h, (((0,), (0,)), ((), ())),
                              preferred_element_type=jnp.float32,
                              precision=_HI)

    @pl.when(j == 0)
    def _():
        g_acc[...] = contrib

    @pl.when(j > 0)
    def _():
        g_acc[...] += contrib

    @pl.when(j == pl.num_programs(0) - 1)
    def _():
        g = jnp.maximum(
            jnp.dot(_r(g_acc[...]), wr1_ref[...],
                    preferred_element_type=jnp.float32, precision=_HI)
            + br1_ref[...], 0.0)
        out_ref[...] = (
            jnp.dot(_r(g), wr2_ref[...], preferred_element_type=jnp.float32,
                    precision=_HI)
            + br2_ref[...])


def _pool_mlp(h3_raw, batch2d, b2, wr1, br1, wr2, br2):
    return pl.pallas_call(
        _pool_mlp_body,
        grid=(N // _BN,),
        in_specs=[
            pl.BlockSpec((_BN, 64), lambda i: (i, 0)),
            pl.BlockSpec((_BN, 1), lambda i: (i, 0)),
            pl.BlockSpec((1, 64), lambda i: (0, 0)),
            pl.BlockSpec((64, 64), lambda i: (0, 0)),
            pl.BlockSpec((1, 64), lambda i: (0, 0)),
            pl.BlockSpec((64, 1), lambda i: (0, 0)),
            pl.BlockSpec((1, 1), lambda i: (0, 0)),
        ],
        out_specs=pl.BlockSpec((NG, 1), lambda i: (0, 0)),
        out_shape=jax.ShapeDtypeStruct((NG, 1), jnp.float32),
        scratch_shapes=[pltpu.VMEM((NG, 64), jnp.float32)],
    )(h3_raw, batch2d, b2, wr1, br1, wr2, br2)


def kernel(x, edge_attr, recipe, params, edge_index, batch):
    p = params
    w_in = p['W_in']
    t0 = jnp.matmul(_r(p['node_emb0']), _r(w_in[8:16]), precision=_HI)
    t1 = jnp.matmul(_r(p['node_emb1']), _r(w_in[16:32]), precision=_HI)
    r8 = jnp.matmul(_r(recipe), _r(w_in[32:48]), precision=_HI)
    w8 = _r(w_in[:8])
    b_in = p['b_in'].reshape(1, 64)

    ve_cols = []
    for i in range(3):
        we = _r(p['l%d_We' % i])
        ae = p['l%d_aedge' % i][0]
        ve_cols.append((we.reshape(64, 4, 16) * ae[None]).sum(-1))
    ve = jnp.concatenate(ve_cols, axis=1)
    m4 = jnp.matmul(_r(p['W_ee'][:4]), ve, precision=_HI)
    te = jnp.matmul(jnp.matmul(_r(p['edge_emb0']), _r(p['W_ee'][4:10]),
                               precision=_HI), ve, precision=_HI)
    ce = jnp.matmul(p['b_ee'].reshape(1, 64), ve, precision=_HI).reshape(1, 12)

    s_mat = (jnp.arange(64)[:, None] // 16
             == jnp.arange(4)[None, :]).astype(jnp.float32)

    batch2d = batch.reshape(N, 1)
    src = edge_index[0]
    dst = edge_index[1]

    h_raw = _node_frontend(x, batch2d, t0, t1, r8, w8, b_in)
    ae_all = _edge_frontend(edge_attr, m4, te, ce)

    zeros_b = jnp.zeros((1, 64), jnp.float32)
    biases = [zeros_b, p['l0_b'].reshape(1, 64), p['l1_b'].reshape(1, 64)]

    for i in range(3):
        dense = _layer_dense(h_raw, biases[i], p['l%d_W' % i],
                             p['l%d_asrc' % i].reshape(1, 64),
                             p['l%d_adst' % i].reshape(1, 64), s_mat)
        hl = dense[:, :64]
        a_src = dense[:, 64:68]
        a_dst = dense[:, 68:72]
        alpha = a_src[src] + a_dst[dst] + ae_all[:, 4 * i:4 * i + 4]
        alpha = jnp.where(alpha >= 0, alpha, 0.2 * alpha)
        ex = jnp.exp(alpha)
        denom = jax.ops.segment_sum(ex, dst, num_segments=N)
        msg = hl.reshape(N, HEADS, CH)[src] * ex[:, :, None]
        agg = jax.ops.segment_sum(msg, dst, num_segments=N)
        h_raw = (agg / (denom[:, :, None] + 1e-16)).reshape(N, 64)

    return _pool_mlp(h_raw, batch2d, p['l2_b'].reshape(1, 64),
                     p['W_r1'], p['b_r1'].reshape(1, 64),
                     p['W_r2'], p['b_r2'].reshape(1, 1))


# speedup vs baseline: 23.5918x; 22.2968x over previous
import functools

import jax
import jax.numpy as jnp
from jax import lax
from jax.experimental import pallas as pl
from jax.experimental.pallas import tpu as pltpu
from jax.experimental.pallas import tpu_sc as plsc

N = 50000
E = 800000
HEADS = 4
NG = 8

NW = 32
TILE_N = 1568
N_PAD = NW * TILE_N
ECH = E // NW
C = 128
E_PAD = E + NW * 8 + C

_BN = TILE_N
_BE = 4000

_HI = lax.Precision.HIGHEST

_sc_mesh = plsc.VectorSubcoreMesh(core_axis_name="c", subcore_axis_name="s")


def _r(v):
    return v.astype(jnp.bfloat16).astype(jnp.float32)


def _wid():
    return lax.axis_index("s") * 2 + lax.axis_index("c")


def _iota16():
    return lax.iota(jnp.int32, 16)


def _full16(v):
    return jnp.full((16,), v, jnp.int32)


def _node_frontend_body(x_ref, batch_ref, t0_ref, t1_ref, r8_ref, w8_ref,
                        b_ref, out_ref):
    xb = x_ref[...]
    x8 = _r(xb[:, 0:8])
    i0 = xb[:, 8:9].astype(jnp.int32)
    i1 = xb[:, 9:10].astype(jnp.int32)
    bb = batch_ref[...]
    bn = xb.shape[0]
    acc = jnp.dot(x8, w8_ref[...], preferred_element_type=jnp.float32,
                  precision=_HI)
    oh0 = (lax.broadcasted_iota(jnp.int32, (bn, 64), 1) == i0).astype(jnp.float32)
    acc += jnp.dot(oh0, t0_ref[...], preferred_element_type=jnp.float32,
                   precision=_HI)
    oh1 = (lax.broadcasted_iota(jnp.int32, (bn, 512), 1) == i1).astype(jnp.float32)
    acc += jnp.dot(oh1, t1_ref[...], preferred_element_type=jnp.float32,
                   precision=_HI)
    ohb = (lax.broadcasted_iota(jnp.int32, (bn, NG), 1) == bb).astype(jnp.float32)
    acc += jnp.dot(ohb, r8_ref[...], preferred_element_type=jnp.float32,
                   precision=_HI)
    out_ref[...] = jnp.maximum(acc + b_ref[...], 0.0)


def _node_frontend(x, batch2d, t0, t1, r8, w8, b):
    return pl.pallas_call(
        _node_frontend_body,
        grid=(N_PAD // _BN,),
        in_specs=[
            pl.BlockSpec((_BN, 10), lambda i: (i, 0)),
            pl.BlockSpec((_BN, 1), lambda i: (i, 0)),
            pl.BlockSpec((64, 64), lambda i: (0, 0)),
            pl.BlockSpec((512, 64), lambda i: (0, 0)),
            pl.BlockSpec((NG, 64), lambda i: (0, 0)),
            pl.BlockSpec((8, 64), lambda i: (0, 0)),
            pl.BlockSpec((1, 64), lambda i: (0, 0)),
        ],
        out_specs=pl.BlockSpec((_BN, 64), lambda i: (i, 0)),
        out_shape=jax.ShapeDtypeStruct((N_PAD, 64), jnp.float32),
    )(x, batch2d, t0, t1, r8, w8, b)


def _edge_frontend_body(ea_ref, m4_ref, te_ref, ce_ref, out_ref):
    eb = ea_ref[...]
    e4 = _r(eb[:, 0:4])
    ei = eb[:, 4:5].astype(jnp.int32)
    be = eb.shape[0]
    acc = jnp.dot(e4, m4_ref[...], preferred_element_type=jnp.float32,
                  precision=_HI)
    ohe = (lax.broadcasted_iota(jnp.int32, (be, 32), 1) == ei).astype(jnp.float32)
    acc += jnp.dot(ohe, te_ref[...], preferred_element_type=jnp.float32,
                   precision=_HI)
    out_ref[...] = jnp.concatenate(
        [acc + ce_ref[...], jnp.zeros((be, 4), jnp.float32)], axis=1)


def _edge_frontend(edge_attr, m4, te, ce):
    return pl.pallas_call(
        _edge_frontend_body,
        grid=(E // _BE,),
        in_specs=[
            pl.BlockSpec((_BE, 5), lambda i: (i, 0)),
            pl.BlockSpec((4, 12), lambda i: (0, 0)),
            pl.BlockSpec((32, 12), lambda i: (0, 0)),
            pl.BlockSpec((1, 12), lambda i: (0, 0)),
        ],
        out_specs=pl.BlockSpec((_BE, 16), lambda i: (i, 0)),
        out_shape=jax.ShapeDtypeStruct((E, 16), jnp.float32),
    )(edge_attr, m4, te, ce)


def _layer_dense_body(h_ref, bprev_ref, w_ref, asrc_ref, adst_ref, s_ref,
                      hl80_ref, ad_ref):
    h = jnp.maximum(h_ref[...] + bprev_ref[...], 0.0)
    hl = jnp.dot(h, w_ref[...], preferred_element_type=jnp.float32)
    a_src = jnp.dot(hl * asrc_ref[...], s_ref[...],
                    preferred_element_type=jnp.float32, precision=_HI)
    a_dst = jnp.dot(hl * adst_ref[...], s_ref[...],
                    preferred_element_type=jnp.float32, precision=_HI)
    bn = h.shape[0]
    hl80_ref[...] = jnp.concatenate(
        [hl, a_src, jnp.zeros((bn, 60), jnp.float32)], axis=1)
    ad_ref[...] = a_dst


def _layer_dense(h_raw, bprev, w, asrc_row, adst_row, s_mat):
    return pl.pallas_call(
        _layer_dense_body,
        grid=(N_PAD // _BN,),
        in_specs=[
            pl.BlockSpec((_BN, 64), lambda i: (i, 0)),
            pl.BlockSpec((1, 64), lambda i: (0, 0)),
            pl.BlockSpec((64, 64), lambda i: (0, 0)),
            pl.BlockSpec((1, 64), lambda i: (0, 0)),
            pl.BlockSpec((1, 64), lambda i: (0, 0)),
            pl.BlockSpec((64, 4), lambda i: (0, 0)),
        ],
        out_specs=[
            pl.BlockSpec((_BN, 128), lambda i: (i, 0)),
            pl.BlockSpec((_BN, 4), lambda i: (i, 0)),
        ],
        out_shape=[
            jax.ShapeDtypeStruct((N_PAD, 128), jnp.float32),
            jax.ShapeDtypeStruct((N_PAD, 4), jnp.float32),
        ],
    )(h_raw, bprev, w, asrc_row, adst_row, s_mat)


def _pool_mlp_body(h_ref, batch_ref, b2_ref, wr1_ref, br1_ref, wr2_ref,
                   br2_ref, out_ref, g_acc):
    j = pl.program_id(0)
    h = jnp.maximum(h_ref[...] + b2_ref[...], 0.0)
    bb = batch_ref[...]
    bn = h.shape[0]
    oh = (lax.broadcasted_iota(jnp.int32, (bn, NG), 1) == bb).astype(jnp.float32)
    contrib = lax.dot_general(oh, h, (((0,), (0,)), ((), ())),
                              preferred_element_type=jnp.float32,
                              precision=_HI)

    @pl.when(j == 0)
    def _():
        g_acc[...] = contrib

    @pl.when(j > 0)
    def _():
        g_acc[...] += contrib

    @pl.when(j == pl.num_programs(0) - 1)
    def _():
        g = jnp.maximum(
            jnp.dot(_r(g_acc[...]), wr1_ref[...],
                    preferred_element_type=jnp.float32, precision=_HI)
            + br1_ref[...], 0.0)
        out_ref[...] = (
            jnp.dot(_r(g), wr2_ref[...], preferred_element_type=jnp.float32,
                    precision=_HI)
            + br2_ref[...])


def _pool_mlp(h3_raw, batch2d, b2, wr1, br1, wr2, br2):
    return pl.pallas_call(
        _pool_mlp_body,
        grid=(N_PAD // _BN,),
        in_specs=[
            pl.BlockSpec((_BN, 64), lambda i: (i, 0)),
            pl.BlockSpec((_BN, 1), lambda i: (i, 0)),
            pl.BlockSpec((1, 64), lambda i: (0, 0)),
            pl.BlockSpec((64, 64), lambda i: (0, 0)),
            pl.BlockSpec((1, 64), lambda i: (0, 0)),
            pl.BlockSpec((64, 1), lambda i: (0, 0)),
            pl.BlockSpec((1, 1), lambda i: (0, 0)),
        ],
        out_specs=pl.BlockSpec((NG, 1), lambda i: (0, 0)),
        out_shape=jax.ShapeDtypeStruct((NG, 1), jnp.float32),
        scratch_shapes=[pltpu.VMEM((NG, 64), jnp.float32)],
    )(h3_raw, batch2d, b2, wr1, br1, wr2, br2)


_NCH_PERM = (ECH + C - 1) // C
_EBUF = _NCH_PERM * C + 16


def _lane0():
    return _iota16() == 0


@functools.partial(
    pl.kernel, mesh=_sc_mesh,
    compiler_params=pltpu.CompilerParams(needs_layout_passes=False),
    out_type=jax.ShapeDtypeStruct((NW * 32,), jnp.int32),
    scratch_types=[
        pltpu.VMEM((_EBUF,), jnp.int32),
        pltpu.VMEM((48,), jnp.int32),
        pltpu.SemaphoreType.DMA,
    ])
def _hist_kernel(dst_hbm, out_hbm, dbuf, cnts, sem):
    w = _wid()
    pltpu.async_copy(dst_hbm.at[pl.ds(pl.multiple_of(w * ECH, 8), ECH)],
                     dbuf.at[pl.ds(0, ECH)], sem).wait()
    zi = jnp.zeros((16,), jnp.int32)
    cnts[pl.ds(0, 16)] = zi
    cnts[pl.ds(16, 16)] = zi
    cnts[pl.ds(32, 16)] = zi
    iota = _iota16()
    one = jnp.full((16,), 1, jnp.int32)

    def vec_body(g, _):
        lanes = iota + g * 16
        d = jnp.maximum(dbuf[pl.ds(g * 16, 16)], 0)
        b = jnp.minimum(lax.div(d, jnp.full((16,), TILE_N, jnp.int32)), NW - 1)
        plsc.addupdate_scatter(cnts, [b], one, mask=lanes < ECH)
        return 0

    lax.fori_loop(0, (ECH + 15) // 16, vec_body, 0)
    pltpu.sync_copy(cnts.at[pl.ds(0, 32)],
                    out_hbm.at[pl.ds(pl.multiple_of(w * 32, 8), 32)])


@functools.partial(
    pl.kernel, mesh=_sc_mesh,
    compiler_params=pltpu.CompilerParams(needs_layout_passes=False),
    out_type=(jax.ShapeDtypeStruct((E_PAD,), jnp.int32),
              jax.ShapeDtypeStruct((E_PAD,), jnp.int32),
              jax.ShapeDtypeStruct((E_PAD,), jnp.int32)),
    scratch_types=[
        pltpu.VMEM((_EBUF,), jnp.int32),
        pltpu.VMEM((_EBUF,), jnp.int32),
        pltpu.VMEM((_EBUF,), jnp.int32),
        pltpu.VMEM((32,), jnp.int32),
        pltpu.SMEM((32,), jnp.int32),
        pltpu.VMEM((C,), jnp.int32),
        pltpu.VMEM((C,), jnp.int32),
        pltpu.VMEM((C,), jnp.int32),
        pltpu.VMEM((C,), jnp.int32),
        pltpu.SemaphoreType.DMA,
    ])
def _permute_kernel(src_hbm, dst_hbm, wstart_hbm, srcp_hbm, dstp_hbm,
                    eidp_hbm, dbuf, bbuf, sbuf, wbuf, ctr, svc, dvc, evc,
                    posb, sem):
    w = _wid()
    base = pl.multiple_of(w * ECH, 8)
    pltpu.async_copy(dst_hbm.at[pl.ds(base, ECH)],
                     dbuf.at[pl.ds(0, ECH)], sem).wait()
    pltpu.async_copy(src_hbm.at[pl.ds(base, ECH)],
                     sbuf.at[pl.ds(0, ECH)], sem).wait()
    pltpu.sync_copy(wstart_hbm.at[pl.ds(pl.multiple_of(w * 32, 8), 32)], wbuf)
    for b in range(NW):
        ctr[b] = plsc.load_gather(wbuf, [_full16(b)])[0]

    iota = _iota16()

    def vec_body(g, _):
        d = jnp.maximum(dbuf[pl.ds(g * 16, 16)], 0)
        bbuf[pl.ds(g * 16, 16)] = jnp.minimum(
            lax.div(d, jnp.full((16,), TILE_N, jnp.int32)), NW - 1)
        return 0

    lax.fori_loop(0, (_NCH_PERM * C) // 16, vec_body, 0)

    lane0 = _lane0()

    def chunk(k, _):
        start = k * C
        real = jnp.minimum(C, ECH - start)

        def group(g, _):
            off = start + g * 16
            svc[pl.ds(g * 16, 16)] = sbuf[pl.ds(off, 16)]
            dvc[pl.ds(g * 16, 16)] = dbuf[pl.ds(off, 16)]
            evc[pl.ds(g * 16, 16)] = jnp.full((16,), base + off,
                                              jnp.int32) + iota
            bv = bbuf[pl.ds(off, 16)]
            for i in range(16):
                li = g * 16 + i
                b = bv[i]
                valid = li < real
                pos = ctr[b]
                ctr[b] = pos + jnp.where(valid, 1, 0)
                pose = jnp.where(valid, pos, E_PAD - 1)
                plsc.store_scatter(posb, [_full16(li)],
                                   jnp.full((16,), pose, jnp.int32),
                                   mask=lane0)
            return 0

        lax.fori_loop(0, C // 16, group, 0)
        c1 = pltpu.async_copy(svc, srcp_hbm.at[posb], sem)
        c2 = pltpu.async_copy(dvc, dstp_hbm.at[posb], sem)
        c3 = pltpu.async_copy(evc, eidp_hbm.at[posb], sem)
        c1.wait()
        c2.wait()
        c3.wait()
        return 0

    lax.fori_loop(0, _NCH_PERM, chunk, 0)


CK = 96


def _make_edge_layer(layer):
    @functools.partial(
        pl.kernel, mesh=_sc_mesh,
        compiler_params=pltpu.CompilerParams(needs_layout_passes=False),
        out_type=jax.ShapeDtypeStruct((N_PAD * 64,), jnp.float32),
        scratch_types=[
            pltpu.VMEM((TILE_N * 64,), jnp.float32),
            pltpu.VMEM((TILE_N * 4,), jnp.float32),
            pltpu.VMEM((TILE_N * 4,), jnp.float32),
            pltpu.VMEM((CK, 128), jnp.float32),
            pltpu.VMEM((4, CK), jnp.float32),
            pltpu.VMEM((CK,), jnp.int32),
            pltpu.VMEM((CK,), jnp.int32),
            pltpu.VMEM((CK,), jnp.int32),
            pltpu.VMEM((CK,), jnp.int32),
            pltpu.VMEM((CK,), jnp.int32),
            pltpu.VMEM((CK,), jnp.int32),
            pltpu.VMEM((CK,), jnp.int32),
            pltpu.VMEM((32,), jnp.int32),
            pltpu.VMEM((32,), jnp.int32),
            pltpu.SemaphoreType.DMA,
        ])
    def _edge_layer(hl_hbm, adst_hbm, srcp_hbm, dstp_hbm, eidp_hbm,
                    ae_hbm, astart_hbm, tot_hbm, out_hbm, tbl, den,
                    adstb, hlb, aeb, srcb, dstb, eidb, ai0, ai1, ai2,
                    ai3, astb, totb, sem):
        w = _wid()
        nb = w * TILE_N
        pltpu.sync_copy(astart_hbm, astb)
        pltpu.sync_copy(tot_hbm, totb)
        pltpu.async_copy(adst_hbm.at[pl.ds(pl.multiple_of(nb * 4, 8),
                                           TILE_N * 4)],
                         adstb, sem).wait()
        ast = plsc.load_gather(astb, [_full16(w)])[0]
        cnt = plsc.load_gather(totb, [_full16(w)])[0]

        zv = jnp.zeros((16,), jnp.float32)

        def z1(g, _):
            tbl[pl.ds(g * 16, 16)] = zv
            return 0

        lax.fori_loop(0, TILE_N * 4, z1, 0)

        def z2(g, _):
            den[pl.ds(g * 16, 16)] = zv
            return 0

        lax.fori_loop(0, (TILE_N * 4) // 16, z2, 0)

        iota = _iota16()
        nch = (cnt + CK - 1) // CK

        def chunk(k, _):
            start = pl.multiple_of(ast + k * CK, 8)
            real = cnt - k * CK
            l1 = pltpu.async_copy(srcp_hbm.at[pl.ds(start, CK)], srcb, sem)
            l2 = pltpu.async_copy(dstp_hbm.at[pl.ds(start, CK)], dstb, sem)
            l3 = pltpu.async_copy(eidp_hbm.at[pl.ds(start, CK)], eidb, sem)
            l1.wait()
            l2.wait()
            l3.wait()

            def prep(g, _):
                sl = pl.ds(g * 16, 16)
                srcb[sl] = jnp.minimum(jnp.maximum(srcb[sl], 0), N - 1)
                dstb[sl] = jnp.minimum(jnp.maximum(dstb[sl] - nb, 0),
                                       TILE_N - 1)
                e16 = jnp.minimum(jnp.maximum(eidb[sl], 0), E - 1) * 16
                ai0[sl] = e16 + (4 * layer)
                ai1[sl] = e16 + (4 * layer + 1)
                ai2[sl] = e16 + (4 * layer + 2)
                ai3[sl] = e16 + (4 * layer + 3)
                return 0

            lax.fori_loop(0, CK // 16, prep, 0)
            g0 = pltpu.async_copy(hl_hbm.at[srcb], hlb, sem)
            g1 = pltpu.async_copy(ae_hbm.at[ai0], aeb.at[0], sem)
            g2 = pltpu.async_copy(ae_hbm.at[ai1], aeb.at[1], sem)
            g3 = pltpu.async_copy(ae_hbm.at[ai2], aeb.at[2], sem)
            g4 = pltpu.async_copy(ae_hbm.at[ai3], aeb.at[3], sem)
            g0.wait()
            g1.wait()
            g2.wait()
            g3.wait()
            g4.wait()

            def group(g, _):
                lanes = iota + g * 16
                valid = lanes < real
                dstl = dstb[pl.ds(g * 16, 16)]
                d64 = dstl * 64
                for h in range(HEADS):
                    a_s = plsc.load_gather(hlb, [lanes, _full16(64 + h)])
                    a_d = plsc.load_gather(adstb, [dstl * 4 + h])
                    a_e = aeb[h, pl.ds(g * 16, 16)]
                    al = a_s + a_d + a_e
                    al = jnp.where(al >= 0.0, al, al * 0.2)
                    exv = jnp.where(valid, jnp.exp(al), 0.0)
                    plsc.addupdate_scatter(den, [dstl * 4 + h], exv)
                    for cc in range(16):
                        val = plsc.load_gather(
                            hlb, [lanes, _full16(h * 16 + cc)]) * exv
                        plsc.addupdate_scatter(
                            tbl, [d64 + (h * 16 + cc)], val)
                return 0

            lax.fori_loop(0, CK // 16, group, 0)
            return 0

        lax.fori_loop(0, nch, chunk, 0)

        def rphase(g, _):
            v = den[pl.ds(g * 16, 16)]
            den[pl.ds(g * 16, 16)] = 1.0 / (v + 1e-16)
            return 0

        lax.fori_loop(0, (TILE_N * 4) // 16, rphase, 0)

        def nphase(n, _):
            row = n * 64
            for h in range(HEADS):
                rvec = plsc.load_gather(den, [jnp.full((16,), n * 4 + h,
                                                       jnp.int32)])
                v = tbl[pl.ds(row + h * 16, 16)]
                tbl[pl.ds(row + h * 16, 16)] = v * rvec
            return 0

        lax.fori_loop(0, TILE_N, nphase, 0)
        pltpu.sync_copy(tbl,
                        out_hbm.at[pl.ds(pl.multiple_of(nb * 64, 8),
                                         TILE_N * 64)])

    return _edge_layer


_edge_layers = [_make_edge_layer(i) for i in range(3)]


def kernel(x, edge_attr, recipe, params, edge_index, batch):
    p = params
    w_in = p['W_in']
    t0 = jnp.matmul(_r(p['node_emb0']), _r(w_in[8:16]), precision=_HI)
    t1 = jnp.matmul(_r(p['node_emb1']), _r(w_in[16:32]), precision=_HI)
    r8 = jnp.matmul(_r(recipe), _r(w_in[32:48]), precision=_HI)
    w8 = _r(w_in[:8])
    b_in = p['b_in'].reshape(1, 64)

    ve_cols = []
    for i in range(3):
        we = _r(p['l%d_We' % i])
        ae = p['l%d_aedge' % i][0]
        ve_cols.append((we.reshape(64, 4, 16) * ae[None]).sum(-1))
    ve = jnp.concatenate(ve_cols, axis=1)
    m4 = jnp.matmul(_r(p['W_ee'][:4]), ve, precision=_HI)
    te = jnp.matmul(jnp.matmul(_r(p['edge_emb0']), _r(p['W_ee'][4:10]),
                               precision=_HI), ve, precision=_HI)
    ce = jnp.matmul(p['b_ee'].reshape(1, 64), ve, precision=_HI).reshape(1, 12)

    s_mat = (jnp.arange(64)[:, None] // 16
             == jnp.arange(4)[None, :]).astype(jnp.float32)

    pad = N_PAD - N
    x_pad = jnp.concatenate([x, jnp.zeros((pad, 10), jnp.float32)], axis=0)
    batch_pad = jnp.concatenate(
        [batch, jnp.full((pad,), NG, jnp.int32)]).reshape(N_PAD, 1)

    src = edge_index[0]
    dst = edge_index[1]

    h_raw = _node_frontend(x_pad, batch_pad, t0, t1, r8, w8, b_in)
    ae16 = _edge_frontend(edge_attr, m4, te, ce)

    counts = _hist_kernel(dst).reshape(NW, 32)
    tot = counts.sum(0, dtype=jnp.int32)
    sz = ((tot + 7) // 8) * 8
    astart = jnp.concatenate(
        [jnp.zeros((1,), jnp.int32), jnp.cumsum(sz, dtype=jnp.int32)])[:32]
    wstart = astart[None, :] + (jnp.cumsum(counts, 0, dtype=jnp.int32)
                                - counts)
    srcp, dstp, eidp = _permute_kernel(src, dst, wstart.reshape(-1))
    ae_flat = ae16.reshape(-1)

    zeros_b = jnp.zeros((1, 64), jnp.float32)
    biases = [zeros_b, p['l0_b'].reshape(1, 64), p['l1_b'].reshape(1, 64)]

    for i in range(3):
        hl80, adst = _layer_dense(h_raw, biases[i], p['l%d_W' % i],
                                  p['l%d_asrc' % i].reshape(1, 64),
                                  p['l%d_adst' % i].reshape(1, 64), s_mat)
        h_agg = _edge_layers[i](hl80, adst.reshape(-1), srcp, dstp,
                                eidp, ae_flat, astart, tot)
        h_raw = h_agg.reshape(N_PAD, 64)

    return _pool_mlp(h_raw, batch_pad, p['l2_b'].reshape(1, 64),
                     p['W_r1'], p['b_r1'].reshape(1, 64),
                     p['W_r2'], p['b_r2'].reshape(1, 1))
